# baseline (speedup 1.0000x reference)
import functools

import jax
import jax.numpy as jnp
from jax import lax
from jax.experimental import pallas as pl
from jax.experimental.pallas import tpu as pltpu
from jax.experimental.pallas import tpu_sc as plsc

SC_ROWS = 4096
TC_BLOCK_ROWS = 512

NUM_CORES = 2
NUM_SUBCORES = 16
NUM_WORKERS = NUM_CORES * NUM_SUBCORES
LANES = 16
CHUNK_ROWS = 4
UNROLL = 4

LN2 = 0.6931471805599453
_LN_POLY_HI2LO = (
    -0.01720806112107329,
    0.18497517510136072,
    -0.8555376323113955,
    2.2311505360242627,
    -3.6488345595695315,
    4.204532967260098,
    -2.099074917831667,
)


def _kl16(t, x):
    bits = plsc.bitcast(t, jnp.int32)
    e = (bits >> 23) - 127
    m = plsc.bitcast((bits & 0x007FFFFF) | 0x3F800000, jnp.float32)
    p = jnp.float32(_LN_POLY_HI2LO[0])
    for c in _LN_POLY_HI2LO[1:]:
        p = p * m + jnp.float32(c)
    ln_t = p + e.astype(jnp.float32) * jnp.float32(LN2)
    return t * ln_t - t * x


def _make_sc_call(cols):
    chunk = CHUNK_ROWS * cols
    rows_per_worker = SC_ROWS // NUM_WORKERS
    n_chunks = rows_per_worker // CHUNK_ROWS
    worker_elems = rows_per_worker * cols

    def body(x_hbm, t_hbm, out_hbm, xb, tb, acc_ref, sx0, sx1, st0, st1):
        wid = lax.axis_index("s") * NUM_CORES + lax.axis_index("c")
        base = wid * worker_elems

        def start(c, slot, sx, st):
            off = base + c * chunk
            pltpu.async_copy(
                x_hbm.at[pl.ds(off, chunk)], xb.at[pl.ds(slot * chunk, chunk)], sx
            )
            pltpu.async_copy(
                t_hbm.at[pl.ds(off, chunk)], tb.at[pl.ds(slot * chunk, chunk)], st
            )

        def wait(c, slot, sx, st):
            off = base + c * chunk
            pltpu.make_async_copy(
                x_hbm.at[pl.ds(off, chunk)], xb.at[pl.ds(slot * chunk, chunk)], sx
            ).wait()
            pltpu.make_async_copy(
                t_hbm.at[pl.ds(off, chunk)], tb.at[pl.ds(slot * chunk, chunk)], st
            ).wait()

        def compute(slot, acc):
            def vbody(v, a):
                o = slot * chunk + v * (LANES * UNROLL)
                for u in range(UNROLL):
                    tt = tb[pl.ds(o + u * LANES, LANES)]
                    xx = xb[pl.ds(o + u * LANES, LANES)]
                    a = a + _kl16(tt, xx)
                return a

            return lax.fori_loop(0, chunk // (LANES * UNROLL), vbody, acc)

        start(0, 0, sx0, st0)
        start(1, 1, sx1, st1)

        def pair_body(p, acc):
            c0 = p * 2
            wait(c0, 0, sx0, st0)
            acc = compute(0, acc)

            @pl.when(c0 + 2 < n_chunks)
            def _():
                start(c0 + 2, 0, sx0, st0)

            wait(c0 + 1, 1, sx1, st1)
            acc = compute(1, acc)

            @pl.when(c0 + 3 < n_chunks)
            def _():
                start(c0 + 3, 1, sx1, st1)

            return acc

        acc = lax.fori_loop(
            0, n_chunks // 2, pair_body, jnp.zeros((LANES,), jnp.float32)
        )
        acc_ref[...] = acc
        pltpu.sync_copy(acc_ref, out_hbm.at[wid])

    return pl.kernel(
        body,
        out_type=jax.ShapeDtypeStruct((NUM_WORKERS, LANES), jnp.float32),
        mesh=plsc.VectorSubcoreMesh(core_axis_name="c", subcore_axis_name="s"),
        compiler_params=pltpu.CompilerParams(needs_layout_passes=False),
        scratch_types=[
            pltpu.VMEM((2 * chunk,), jnp.float32),
            pltpu.VMEM((2 * chunk,), jnp.float32),
            pltpu.VMEM((LANES,), jnp.float32),
            pltpu.SemaphoreType.DMA,
            pltpu.SemaphoreType.DMA,
            pltpu.SemaphoreType.DMA,
            pltpu.SemaphoreType.DMA,
        ],
    )


def _tc_kl_sum_kernel(x_ref, t_ref, o_ref):
    i = pl.program_id(0)
    t = t_ref[...]
    x = x_ref[...]
    safe_t = jnp.where(t > 0, t, 1.0)
    kl = t * jnp.log(safe_t) - t * x
    s = jnp.sum(kl)

    @pl.when(i == 0)
    def _init():
        o_ref[0, 0] = 0.0

    o_ref[0, 0] += s


def kernel(inputs, targets):
    rows, cols = inputs.shape
    tc_rows = rows - SC_ROWS
    grid = tc_rows // TC_BLOCK_ROWS
    row_off = SC_ROWS // TC_BLOCK_ROWS

    tc_sum = pl.pallas_call(
        _tc_kl_sum_kernel,
        grid=(grid,),
        in_specs=[
            pl.BlockSpec((TC_BLOCK_ROWS, cols), lambda i: (i + row_off, 0)),
            pl.BlockSpec((TC_BLOCK_ROWS, cols), lambda i: (i + row_off, 0)),
        ],
        out_specs=pl.BlockSpec((1, 1), lambda i: (0, 0), memory_space=pltpu.SMEM),
        out_shape=jax.ShapeDtypeStruct((1, 1), jnp.float32),
        compiler_params=pltpu.CompilerParams(
            dimension_semantics=("arbitrary",),
        ),
    )(inputs, targets)

    sc_partials = _make_sc_call(cols)(
        inputs.reshape(-1), targets.reshape(-1)
    )

    total = tc_sum[0, 0] + jnp.sum(sc_partials)
    return (total / (rows * cols)).astype(jnp.float32)


# speedup vs baseline: 3.0767x; 3.0767x over previous
import jax
import jax.numpy as jnp
from jax import lax
from jax.experimental import pallas as pl
from jax.experimental.pallas import tpu as pltpu
from jax.experimental.pallas import tpu_sc as plsc

SC_ROWS = 4096
TC_BLOCK_ROWS = 512

NUM_CORES = 2
NUM_SUBCORES = 16
NUM_WORKERS = NUM_CORES * NUM_SUBCORES
LANES = 16
CHUNK_ROWS = 4
UNROLL = 4

LN2 = 0.6931471805599453
_Q_HI2LO = (
    -0.07915036575312574,
    0.3122142661721608,
    -0.6695152104236199,
    0.4360980844701042,
    -126.99979584108635,
)


def _kl16(t, x):
    bits = plsc.bitcast(t, jnp.int32)
    s = bits.astype(jnp.float32) * jnp.float32(2.0**-23)
    f = s - (bits >> 23).astype(jnp.float32)
    q = jnp.float32(_Q_HI2LO[0])
    for c in _Q_HI2LO[1:]:
        q = q * f + jnp.float32(c)
    ln_t = (s + q) * jnp.float32(LN2)
    return t * (ln_t - x)


def _make_sc_call(cols):
    rows_per_worker = SC_ROWS // NUM_WORKERS
    n_chunks = rows_per_worker // CHUNK_ROWS
    vecs_per_row = cols // (LANES * UNROLL)

    def body(x_hbm, t_hbm, out_hbm, xb, tb, acc_ref, sx0, sx1, st0, st1):
        wid = lax.axis_index("s") * NUM_CORES + lax.axis_index("c")
        base_row = wid * rows_per_worker

        def start(c, slot, sx, st):
            r = base_row + c * CHUNK_ROWS
            pltpu.async_copy(
                x_hbm.at[pl.ds(r, CHUNK_ROWS)],
                xb.at[pl.ds(slot * CHUNK_ROWS, CHUNK_ROWS)],
                sx,
            )
            pltpu.async_copy(
                t_hbm.at[pl.ds(r, CHUNK_ROWS)],
                tb.at[pl.ds(slot * CHUNK_ROWS, CHUNK_ROWS)],
                st,
            )

        def wait(c, slot, sx, st):
            r = base_row + c * CHUNK_ROWS
            pltpu.make_async_copy(
                x_hbm.at[pl.ds(r, CHUNK_ROWS)],
                xb.at[pl.ds(slot * CHUNK_ROWS, CHUNK_ROWS)],
                sx,
            ).wait()
            pltpu.make_async_copy(
                t_hbm.at[pl.ds(r, CHUNK_ROWS)],
                tb.at[pl.ds(slot * CHUNK_ROWS, CHUNK_ROWS)],
                st,
            ).wait()

        def compute(slot, acc):
            for rr in range(CHUNK_ROWS):
                row = slot * CHUNK_ROWS + rr

                def vbody(v, a, row=row):
                    o = v * (LANES * UNROLL)
                    for u in range(UNROLL):
                        tt = tb[row, pl.ds(o + u * LANES, LANES)]
                        xx = xb[row, pl.ds(o + u * LANES, LANES)]
                        a = a + _kl16(tt, xx)
                    return a

                acc = lax.fori_loop(0, vecs_per_row, vbody, acc)
            return acc

        start(0, 0, sx0, st0)
        start(1, 1, sx1, st1)

        def pair_body(p, acc):
            c0 = p * 2
            wait(c0, 0, sx0, st0)
            acc = compute(0, acc)

            @pl.when(c0 + 2 < n_chunks)
            def _():
                start(c0 + 2, 0, sx0, st0)

            wait(c0 + 1, 1, sx1, st1)
            acc = compute(1, acc)

            @pl.when(c0 + 3 < n_chunks)
            def _():
                start(c0 + 3, 1, sx1, st1)

            return acc

        acc = lax.fori_loop(
            0, n_chunks // 2, pair_body, jnp.zeros((LANES,), jnp.float32)
        )
        acc_ref[...] = acc
        pltpu.sync_copy(acc_ref, out_hbm.at[wid])

    return pl.kernel(
        body,
        out_type=jax.ShapeDtypeStruct((NUM_WORKERS, LANES), jnp.float32),
        mesh=plsc.VectorSubcoreMesh(core_axis_name="c", subcore_axis_name="s"),
        compiler_params=pltpu.CompilerParams(needs_layout_passes=False),
        scratch_types=[
            pltpu.VMEM((2 * CHUNK_ROWS, cols), jnp.float32),
            pltpu.VMEM((2 * CHUNK_ROWS, cols), jnp.float32),
            pltpu.VMEM((LANES,), jnp.float32),
            pltpu.SemaphoreType.DMA,
            pltpu.SemaphoreType.DMA,
            pltpu.SemaphoreType.DMA,
            pltpu.SemaphoreType.DMA,
        ],
    )


def _tc_kl_sum_kernel(x_ref, t_ref, o_ref):
    i = pl.program_id(0)
    t = t_ref[...]
    x = x_ref[...]
    safe_t = jnp.where(t > 0, t, 1.0)
    kl = t * jnp.log(safe_t) - t * x
    s = jnp.sum(kl)

    @pl.when(i == 0)
    def _init():
        o_ref[0, 0] = 0.0

    o_ref[0, 0] += s


def kernel(inputs, targets):
    rows, cols = inputs.shape
    tc_rows = rows - SC_ROWS
    grid = tc_rows // TC_BLOCK_ROWS
    row_off = SC_ROWS // TC_BLOCK_ROWS

    tc_sum = pl.pallas_call(
        _tc_kl_sum_kernel,
        grid=(grid,),
        in_specs=[
            pl.BlockSpec((TC_BLOCK_ROWS, cols), lambda i: (i + row_off, 0)),
            pl.BlockSpec((TC_BLOCK_ROWS, cols), lambda i: (i + row_off, 0)),
        ],
        out_specs=pl.BlockSpec((1, 1), lambda i: (0, 0), memory_space=pltpu.SMEM),
        out_shape=jax.ShapeDtypeStruct((1, 1), jnp.float32),
        compiler_params=pltpu.CompilerParams(
            dimension_semantics=("arbitrary",),
        ),
    )(inputs, targets)

    sc_partials = _make_sc_call(cols)(inputs, targets)

    total = tc_sum[0, 0] + jnp.sum(sc_partials)
    return (total / (rows * cols)).astype(jnp.float32)


# speedup vs baseline: 3.4646x; 1.1261x over previous
import jax
import jax.numpy as jnp
from jax.experimental import pallas as pl
from jax.experimental.pallas import tpu as pltpu

BLOCK_ROWS = 256


def _kl_sum_kernel(x_ref, t_ref, o_ref):
    i = pl.program_id(0)
    t = t_ref[...]
    x = x_ref[...]
    safe_t = jnp.where(t > 0, t, 1.0)
    kl = t * jnp.log(safe_t) - t * x
    s = jnp.sum(kl)

    @pl.when(i == 0)
    def _init():
        o_ref[0, 0] = 0.0

    o_ref[0, 0] += s


def kernel(inputs, targets):
    rows, cols = inputs.shape
    grid = rows // BLOCK_ROWS

    out = pl.pallas_call(
        _kl_sum_kernel,
        grid=(grid,),
        in_specs=[
            pl.BlockSpec((BLOCK_ROWS, cols), lambda i: (i, 0)),
            pl.BlockSpec((BLOCK_ROWS, cols), lambda i: (i, 0)),
        ],
        out_specs=pl.BlockSpec((1, 1), lambda i: (0, 0), memory_space=pltpu.SMEM),
        out_shape=jax.ShapeDtypeStruct((1, 1), jnp.float32),
        compiler_params=pltpu.CompilerParams(
            dimension_semantics=("arbitrary",),
        ),
    )(inputs, targets)
    return (out[0, 0] / (rows * cols)).astype(jnp.float32)


# speedup vs baseline: 3.5059x; 1.0119x over previous
import jax
import jax.numpy as jnp
from jax.experimental import pallas as pl
from jax.experimental.pallas import tpu as pltpu

BLOCK_ROWS = 512


def _make_kl_sum_kernel(grid, inv_n):
    def _kl_sum_kernel(x_ref, t_ref, o_ref):
        i = pl.program_id(0)
        t = t_ref[...]
        x = x_ref[...]
        safe_t = jnp.where(t > 0, t, 1.0)
        kl = t * jnp.log(safe_t) - t * x
        s = jnp.sum(kl)

        @pl.when(i == 0)
        def _init():
            o_ref[0, 0] = 0.0

        o_ref[0, 0] += s

        @pl.when(i == grid - 1)
        def _finalize():
            o_ref[0, 0] *= inv_n

    return _kl_sum_kernel


def kernel(inputs, targets):
    rows, cols = inputs.shape
    grid = rows // BLOCK_ROWS

    out = pl.pallas_call(
        _make_kl_sum_kernel(grid, 1.0 / (rows * cols)),
        grid=(grid,),
        in_specs=[
            pl.BlockSpec((BLOCK_ROWS, cols), lambda i: (i, 0)),
            pl.BlockSpec((BLOCK_ROWS, cols), lambda i: (i, 0)),
        ],
        out_specs=pl.BlockSpec((1, 1), lambda i: (0, 0), memory_space=pltpu.SMEM),
        out_shape=jax.ShapeDtypeStruct((1, 1), jnp.float32),
        compiler_params=pltpu.CompilerParams(
            dimension_semantics=("arbitrary",),
        ),
    )(inputs, targets)
    return out.reshape(())
